# baseline (speedup 1.0000x reference)
import functools

import jax
import jax.numpy as jnp
import numpy as np
from jax import lax
from jax.experimental import pallas as pl
from jax.experimental.pallas import tpu as pltpu
from jax.experimental.pallas import tpu_sc as plsc

_FIELD_DIM = 38462
_NUM_FIELDS = 26
_BATCH = 16384
_EMBED = 16

_NC, _NS = 2, 16
_NW = _NC * _NS
_FLAT = _BATCH * _NUM_FIELDS
_PER_W = _FLAT // _NW
_NCHUNK = 8
_CLEN = _PER_W // _NCHUNK

_OFFS_NP = np.tile(np.arange(_NUM_FIELDS, dtype=np.int32) * _FIELD_DIM,
                   _CLEN // _NUM_FIELDS)

_mesh = plsc.VectorSubcoreMesh(core_axis_name="c", subcore_axis_name="s")


@functools.partial(
    pl.kernel,
    mesh=_mesh,
    out_type=jax.ShapeDtypeStruct((_FLAT, _EMBED), jnp.float32),
    compiler_params=pltpu.CompilerParams(use_tc_tiling_on_sc=False),
    scratch_types=[
        pltpu.VMEM((_PER_W,), jnp.int32),
        pltpu.VMEM((_CLEN,), jnp.int32),
        pltpu.VMEM((_CLEN, _EMBED), jnp.float32),
        pltpu.VMEM((_CLEN, _EMBED), jnp.float32),
        pltpu.SemaphoreType.DMA,
        pltpu.SemaphoreType.DMA,
    ],
)
def _sc_embed(x_hbm, offs_hbm, table_hbm, out_hbm,
              idx_v, offs_v, rows0, rows1, sem0, sem1):
    wid = lax.axis_index("s") * _NC + lax.axis_index("c")
    base = wid * _PER_W

    pltpu.sync_copy(x_hbm.at[pl.ds(base, _PER_W)], idx_v)
    pltpu.sync_copy(offs_hbm, offs_v)

    bufs = (rows0, rows1)
    sems = (sem0, sem1)
    handles = [None, None]

    def add_offsets(chunk):
        def body(j, _):
            sl = pl.ds(chunk * _CLEN + j * 16, 16)
            idx_v[sl] = idx_v[sl] + offs_v[pl.ds(j * 16, 16)]
            return 0
        lax.fori_loop(0, _CLEN // 16, body, 0)

    for i in range(_NCHUNK):
        b = i % 2
        add_offsets(i)
        handles[b] = pltpu.async_copy(
            table_hbm.at[idx_v.at[pl.ds(i * _CLEN, _CLEN)]], bufs[b], sems[b])
        if i > 0:
            pb = (i - 1) % 2
            handles[pb].wait()
            pltpu.sync_copy(
                bufs[pb], out_hbm.at[pl.ds(base + (i - 1) * _CLEN, _CLEN)])
    last = (_NCHUNK - 1) % 2
    handles[last].wait()
    pltpu.sync_copy(
        bufs[last], out_hbm.at[pl.ds(base + (_NCHUNK - 1) * _CLEN, _CLEN)])


def kernel(x, table):
    x1d = x.reshape(_FLAT)
    offs = jnp.asarray(_OFFS_NP)
    out = _sc_embed(x1d, offs, table)
    return out.reshape(_BATCH, _NUM_FIELDS, _EMBED)


# speedup vs baseline: 1.5093x; 1.5093x over previous
import functools

import jax
import jax.numpy as jnp
import numpy as np
from jax import lax
from jax.experimental import pallas as pl
from jax.experimental.pallas import tpu as pltpu
from jax.experimental.pallas import tpu_sc as plsc

_FIELD_DIM = 38462
_NF = 26
_B = 16384
_E = 16
_V = _FIELD_DIM * _NF
_VP = 1000064
_NTC = _VP // 128
_FLAT = _B * _NF
_NW = 32
_BPW = _B // _NW
_CB = 128
_CLEN = _CB * _NF
_NCHUNK = _BPW // _CB

_TCPB = 8
_DTW = _TCPB * 128
_NCB = (_NTC - 1) // _TCPB
_DSTEPS = (_NCB + _NW - 1) // _NW
_TAIL_TC = _NCB * _TCPB

_JJ = np.arange(_CLEN, dtype=np.int32)
_ROWC_NP = _JJ % _NF
_COLC_NP = _JJ // _NF
_OFFC_NP = (_JJ % _NF) * _FIELD_DIM

_mesh = plsc.VectorSubcoreMesh(core_axis_name="c", subcore_axis_name="s")
_TILED = pltpu.CompilerParams(use_tc_tiling_on_sc=True,
                              needs_layout_passes=False)


@functools.partial(
    pl.kernel,
    mesh=_mesh,
    out_type=(jax.ShapeDtypeStruct((_VP * _E,), jnp.float32),
              jax.ShapeDtypeStruct((_FLAT,), jnp.int32)),
    compiler_params=_TILED,
    scratch_types=[
        pltpu.VMEM((16, _DTW), jnp.float32),
        pltpu.VMEM((16, _DTW), jnp.float32),
        pltpu.VMEM((_DTW * 16,), jnp.float32),
        pltpu.VMEM((_DTW * 16,), jnp.float32),
        pltpu.VMEM((_NF, _CB), jnp.int32),
        pltpu.VMEM((_CLEN,), jnp.int32),
        pltpu.VMEM((_CLEN,), jnp.int32),
        pltpu.VMEM((_CLEN,), jnp.int32),
        pltpu.VMEM((_CLEN,), jnp.int32),
        pltpu.VMEM((16, 512), jnp.float32),
        pltpu.VMEM((8192,), jnp.float32),
        pltpu.VMEM((16, 76), jnp.float32),
        pltpu.VMEM((1216,), jnp.float32),
        pltpu.SemaphoreType.DMA,
        pltpu.SemaphoreType.DMA,
        pltpu.SemaphoreType.DMA,
        pltpu.SemaphoreType.DMA,
        pltpu.SemaphoreType.DMA,
    ],
)
def _prep(tt_hbm, xt_hbm, rowc_hbm, colc_hbm, offc_hbm,
          flat_hbm, idx_hbm,
          stage0, stage1, rows0, rows1, xstage, idxbuf,
          rowc_v, colc_v, offc_v, res_s, res_r, tail_s, tail_r,
          rs0, rs1, ws0, ws1, ssem):
    wid = lax.axis_index("s") * 2 + lax.axis_index("c")
    row16 = lax.iota(jnp.int32, 16)
    stages = (stage0, stage1)
    rows = (rows0, rows1)
    rsems = (rs0, rs1)
    wsems = (ws0, ws1)

    pltpu.async_copy(rowc_hbm, rowc_v, ssem)
    pltpu.async_copy(colc_hbm, colc_v, ssem)
    pltpu.async_copy(offc_hbm, offc_v, ssem).wait()
    pltpu.make_async_copy(rowc_hbm, rowc_v, ssem).wait()
    pltpu.make_async_copy(colc_hbm, colc_v, ssem).wait()
    for blk in range(_NCHUNK):
        bsl = pl.ds((wid * _NCHUNK + blk) * _CB, _CB)
        pltpu.async_copy(xt_hbm.at[pl.ds(0, 8), bsl],
                         xstage.at[pl.ds(0, 8)], ssem)
        pltpu.async_copy(xt_hbm.at[pl.ds(8, 8), bsl],
                         xstage.at[pl.ds(8, 8)], ssem)
        pltpu.async_copy(xt_hbm.at[pl.ds(16, 8), bsl],
                         xstage.at[pl.ds(16, 8)], ssem)
        pltpu.async_copy(xt_hbm.at[pl.ds(24, 2), bsl],
                         xstage.at[pl.ds(24, 2)], ssem).wait()
        pltpu.make_async_copy(xt_hbm.at[pl.ds(0, 8), bsl],
                              xstage.at[pl.ds(0, 8)], ssem).wait()
        pltpu.make_async_copy(xt_hbm.at[pl.ds(8, 8), bsl],
                              xstage.at[pl.ds(8, 8)], ssem).wait()
        pltpu.make_async_copy(xt_hbm.at[pl.ds(16, 8), bsl],
                              xstage.at[pl.ds(16, 8)], ssem).wait()

        def xbody(v, _):
            sl = pl.ds(v * 16, 16)
            g = plsc.load_gather(xstage, [rowc_v[sl], colc_v[sl]])
            idxbuf[sl] = g + offc_v[sl]
            return 0
        lax.fori_loop(0, _CLEN // 16, xbody, 0)
        pltpu.sync_copy(
            idxbuf, idx_hbm.at[pl.ds((wid * _NCHUNK + blk) * _CLEN, _CLEN)])

    def rd(cb, b):
        return pltpu.async_copy(
            tt_hbm.at[pl.ds(0, 16), pl.ds(cb * _DTW, _DTW)], stages[b],
            rsems[b])

    def rd_wait(b):
        pltpu.make_async_copy(
            tt_hbm.at[pl.ds(0, 16), pl.ds(0, _DTW)], stages[b],
            rsems[b]).wait()

    def wr(cb, b):
        return pltpu.async_copy(
            rows[b], flat_hbm.at[pl.ds(cb * _DTW * 16, _DTW * 16)], wsems[b])

    def wr_wait(b):
        pltpu.make_async_copy(
            rows[b], flat_hbm.at[pl.ds(0, _DTW * 16)], wsems[b]).wait()

    def transpose_block(stage, rbuf, width):
        def tbody(jj, _):
            for u in range(8):
                j = jj * 8 + u
                g = plsc.load_gather(stage, [row16, row16 * 0 + j])
                rbuf[pl.ds(j * 16, 16)] = g
            return 0
        lax.fori_loop(0, width // 8, tbody, 0)

    cb0 = wid

    @pl.when(cb0 < _NCB)
    def _():
        rd(cb0, 0)

    def dstep(k, _):
        cb = k * _NW + wid
        cbn = cb + _NW

        def body(b, nb):
            @pl.when(cbn < _NCB)
            def _():
                rd(cbn, nb)

            @pl.when(cb < _NCB)
            def _():
                rd_wait(b)

                @pl.when(k >= 2)
                def _():
                    wr_wait(b)
                transpose_block(stages[b], rows[b], _DTW)
                wr(cb, b)

        @pl.when(k % 2 == 0)
        def _():
            body(0, 1)

        @pl.when(k % 2 == 1)
        def _():
            body(1, 0)
        return 0
    lax.fori_loop(0, _DSTEPS, dstep, 0)

    wr_wait(0)
    wr_wait(1)

    @pl.when(wid == _NW - 2)
    def _():
        pltpu.sync_copy(
            tt_hbm.at[pl.ds(0, 16), pl.ds(_TAIL_TC * 128, 512)], res_s)
        transpose_block(res_s, res_r, 512)
        pltpu.sync_copy(
            res_r, flat_hbm.at[pl.ds(_TAIL_TC * 128 * 16, 8192)])

    @pl.when(wid == _NW - 1)
    def _():
        for c in range(16):
            pltpu.sync_copy(
                tt_hbm.at[c, pl.ds((_NTC - 1) * 128, 76)], tail_s.at[c])

        def tailbody(j, _):
            g = plsc.load_gather(tail_s, [row16, row16 * 0 + j])
            tail_r[pl.ds(j * 16, 16)] = g
            return 0
        lax.fori_loop(0, 76, tailbody, 0)
        pltpu.sync_copy(
            tail_r, flat_hbm.at[pl.ds((_NTC - 1) * 128 * 16, 1216)])


@functools.partial(
    pl.kernel,
    mesh=_mesh,
    out_type=jax.ShapeDtypeStruct((_NF * 16, 8, 2048), jnp.float32),
    compiler_params=pltpu.CompilerParams(use_tc_tiling_on_sc=False,
                                         needs_layout_passes=False),
    scratch_types=[
        pltpu.VMEM((_BPW * _NF,), jnp.int32),
        pltpu.VMEM((_CLEN, _E), jnp.float32),
        pltpu.VMEM((_CLEN, _E), jnp.float32),
        pltpu.VMEM((8, _CB), jnp.float32),
        pltpu.VMEM((8, _CB), jnp.float32),
        pltpu.SemaphoreType.DMA,
        pltpu.SemaphoreType.DMA,
        pltpu.SemaphoreType.DMA,
        pltpu.SemaphoreType.DMA,
        pltpu.SemaphoreType.DMA,
    ],
)
def _gather(idx_hbm, rows_hbm, out_hbm,
            idx_v, gb0, gb1, wb0, wb1, sg0, sg1, sw0, sw1, ssem):
    wid = lax.axis_index("s") * 2 + lax.axis_index("c")
    r26 = lax.iota(jnp.int32, 16) * _NF
    zero16 = lax.iota(jnp.int32, 16) * 0
    gbufs = (gb0, gb1)
    gsems = (sg0, sg1)
    wbufs = (wb0, wb1)
    wsems = (sw0, sw1)

    pltpu.sync_copy(
        idx_hbm.at[pl.ds(wid * _BPW * _NF, _BPW * _NF)], idx_v)

    def start_chunk(blk, b):
        return pltpu.async_copy(
            rows_hbm.at[idx_v.at[pl.ds(blk * _CLEN, _CLEN)]], gbufs[b],
            gsems[b])

    def emit_chunk(blk, b):
        gbuf = gbufs[b]
        tcb = wid * _NCHUNK + blk
        u_base = tcb // 16
        col = (tcb % 16) * _CB

        def tbody(t, _):
            f = t >> 1
            tr = t & 1

            def wait_par(wb, ws):
                pltpu.make_async_copy(
                    wb, out_hbm.at[0, pl.ds(0, 8), pl.ds(0, _CB)], ws).wait()

            @pl.when((t >= 2) & (tr == 0))
            def _():
                wait_par(wbufs[0], wsems[0])

            @pl.when((t >= 2) & (tr == 1))
            def _():
                wait_par(wbufs[1], wsems[1])

            def build(wb):
                def dbody(dbb, _):
                    rowvec = r26 + (dbb * (16 * _NF) + f)
                    for cq in range(8):
                        colvec = zero16 + (tr * 8 + cq)
                        g = plsc.load_gather(gbuf, [rowvec, colvec])
                        wb[cq, pl.ds(dbb * 16, 16)] = g
                    return 0
                lax.fori_loop(0, 8, dbody, 0)

            u = f * 16 + tr * 8 + u_base

            @pl.when(tr == 0)
            def _():
                build(wbufs[0])
                pltpu.async_copy(
                    wbufs[0], out_hbm.at[u, pl.ds(0, 8), pl.ds(col, _CB)],
                    wsems[0])

            @pl.when(tr == 1)
            def _():
                build(wbufs[1])
                pltpu.async_copy(
                    wbufs[1], out_hbm.at[u, pl.ds(0, 8), pl.ds(col, _CB)],
                    wsems[1])
            return 0
        lax.fori_loop(0, _NF * 2, tbody, 0)
        pltpu.make_async_copy(
            wbufs[0], out_hbm.at[0, pl.ds(0, 8), pl.ds(0, _CB)],
            wsems[0]).wait()
        pltpu.make_async_copy(
            wbufs[1], out_hbm.at[0, pl.ds(0, 8), pl.ds(0, _CB)],
            wsems[1]).wait()

    h = start_chunk(0, 0)
    for blk in range(_NCHUNK):
        if blk + 1 < _NCHUNK:
            hn = start_chunk(blk + 1, (blk + 1) % 2)
        h.wait()
        emit_chunk(blk, blk % 2)
        if blk + 1 < _NCHUNK:
            h = hn


_NUNIT = _NF * 16
_UPW = _NUNIT // _NW


@functools.partial(
    pl.kernel,
    mesh=_mesh,
    out_type=jax.ShapeDtypeStruct((_NF, _E, _B), jnp.float32),
    compiler_params=_TILED,
    scratch_types=[
        pltpu.VMEM((8, 2048), jnp.float32),
        pltpu.VMEM((8, 2048), jnp.float32),
        pltpu.SemaphoreType.DMA,
        pltpu.SemaphoreType.DMA,
        pltpu.SemaphoreType.DMA,
        pltpu.SemaphoreType.DMA,
    ],
)
def _format(o1d_hbm, out_hbm, st0, st1, rs0, rs1, ws0, ws1):
    wid = lax.axis_index("s") * 2 + lax.axis_index("c")
    stages = (st0, st1)
    rsems = (rs0, rs1)
    wsems = (ws0, ws1)

    def rd(u, b):
        base = u * 16384
        for cq in range(8):
            pltpu.async_copy(
                o1d_hbm.at[pl.ds(base + cq * 2048, 2048)],
                stages[b].at[cq], rsems[b])

    def rd_wait(b):
        for cq in range(8):
            pltpu.make_async_copy(
                o1d_hbm.at[pl.ds(0, 2048)], stages[b].at[cq],
                rsems[b]).wait()

    def wr(u, b):
        f = u // 16
        r8 = u % 16
        tr = r8 // 8
        e = r8 % 8
        return pltpu.async_copy(
            stages[b],
            out_hbm.at[f, pl.ds(tr * 8, 8), pl.ds(e * 2048, 2048)],
            wsems[b])

    def wr_wait(b):
        pltpu.make_async_copy(
            stages[b], out_hbm.at[0, pl.ds(0, 8), pl.ds(0, 2048)],
            wsems[b]).wait()

    rd(wid * _UPW, 0)
    for i in range(_UPW):
        u = wid * _UPW + i
        b = i % 2
        nb = (i + 1) % 2
        if i + 1 < _UPW:
            if i >= 1:
                wr_wait(nb)
            rd(u + 1, nb)
        rd_wait(b)
        wr(u, b)
    wr_wait(0)
    wr_wait(1)


def kernel(x, table):
    tt = jnp.transpose(table)
    xt = jnp.transpose(x)
    rowc = jnp.asarray(_ROWC_NP)
    colc = jnp.asarray(_COLC_NP)
    offc = jnp.asarray(_OFFC_NP)
    flat, idx = _prep(tt, xt, rowc, colc, offc)
    rows2d = flat.reshape(_VP, _E)
    o3 = _gather(idx, rows2d)
    o1d = o3.reshape(_NUNIT * 16384)
    out = _format(o1d)
    return jnp.transpose(out, (2, 0, 1))


# speedup vs baseline: 2.1922x; 1.4524x over previous
import functools

import jax
import jax.numpy as jnp
import numpy as np
from jax import lax
from jax.experimental import pallas as pl
from jax.experimental.pallas import tpu as pltpu
from jax.experimental.pallas import tpu_sc as plsc

_FIELD_DIM = 38462
_NF = 26
_B = 16384
_E = 16
_V = _FIELD_DIM * _NF
_VP = 1000064
_NTC = _VP // 128
_FLAT = _B * _NF
_NW = 32
_BPW = _B // _NW
_CB = 128
_CLEN = _CB * _NF
_NCHUNK = _BPW // _CB

_TCPB = 8
_DTW = _TCPB * 128
_NCB = (_NTC - 1) // _TCPB
_DSTEPS = (_NCB + _NW - 1) // _NW
_TAIL_TC = _NCB * _TCPB

_JJ = np.arange(_CLEN, dtype=np.int32)
_ROWC_NP = _JJ % _NF
_COLC_NP = _JJ // _NF
_OFFC_NP = (_JJ % _NF) * _FIELD_DIM

_mesh = plsc.VectorSubcoreMesh(core_axis_name="c", subcore_axis_name="s")
_TILED = pltpu.CompilerParams(use_tc_tiling_on_sc=True,
                              needs_layout_passes=False)


@functools.partial(
    pl.kernel,
    mesh=_mesh,
    out_type=(jax.ShapeDtypeStruct((_VP * _E,), jnp.float32),
              jax.ShapeDtypeStruct((_FLAT,), jnp.int32)),
    compiler_params=_TILED,
    scratch_types=[
        pltpu.VMEM((16, _DTW), jnp.float32),
        pltpu.VMEM((16, _DTW), jnp.float32),
        pltpu.VMEM((_DTW * 16,), jnp.float32),
        pltpu.VMEM((_DTW * 16,), jnp.float32),
        pltpu.VMEM((_NF, _CB), jnp.int32),
        pltpu.VMEM((_CLEN,), jnp.int32),
        pltpu.VMEM((_CLEN,), jnp.int32),
        pltpu.VMEM((_CLEN,), jnp.int32),
        pltpu.VMEM((_CLEN,), jnp.int32),
        pltpu.VMEM((16, 512), jnp.float32),
        pltpu.VMEM((8192,), jnp.float32),
        pltpu.VMEM((16, 76), jnp.float32),
        pltpu.VMEM((1216,), jnp.float32),
        pltpu.SemaphoreType.DMA,
        pltpu.SemaphoreType.DMA,
        pltpu.SemaphoreType.DMA,
        pltpu.SemaphoreType.DMA,
        pltpu.SemaphoreType.DMA,
    ],
)
def _prep(tt_hbm, xt_hbm, rowc_hbm, colc_hbm, offc_hbm,
          flat_hbm, idx_hbm,
          stage0, stage1, rows0, rows1, xstage, idxbuf,
          rowc_v, colc_v, offc_v, res_s, res_r, tail_s, tail_r,
          rs0, rs1, ws0, ws1, ssem):
    wid = lax.axis_index("s") * 2 + lax.axis_index("c")
    row16 = lax.iota(jnp.int32, 16)
    stages = (stage0, stage1)
    rows = (rows0, rows1)
    rsems = (rs0, rs1)
    wsems = (ws0, ws1)

    pltpu.async_copy(rowc_hbm, rowc_v, ssem)
    pltpu.async_copy(colc_hbm, colc_v, ssem)
    pltpu.async_copy(offc_hbm, offc_v, ssem).wait()
    pltpu.make_async_copy(rowc_hbm, rowc_v, ssem).wait()
    pltpu.make_async_copy(colc_hbm, colc_v, ssem).wait()
    for blk in range(_NCHUNK):
        bsl = pl.ds((wid * _NCHUNK + blk) * _CB, _CB)
        pltpu.async_copy(xt_hbm.at[pl.ds(0, 8), bsl],
                         xstage.at[pl.ds(0, 8)], ssem)
        pltpu.async_copy(xt_hbm.at[pl.ds(8, 8), bsl],
                         xstage.at[pl.ds(8, 8)], ssem)
        pltpu.async_copy(xt_hbm.at[pl.ds(16, 8), bsl],
                         xstage.at[pl.ds(16, 8)], ssem)
        pltpu.async_copy(xt_hbm.at[pl.ds(24, 2), bsl],
                         xstage.at[pl.ds(24, 2)], ssem).wait()
        pltpu.make_async_copy(xt_hbm.at[pl.ds(0, 8), bsl],
                              xstage.at[pl.ds(0, 8)], ssem).wait()
        pltpu.make_async_copy(xt_hbm.at[pl.ds(8, 8), bsl],
                              xstage.at[pl.ds(8, 8)], ssem).wait()
        pltpu.make_async_copy(xt_hbm.at[pl.ds(16, 8), bsl],
                              xstage.at[pl.ds(16, 8)], ssem).wait()

        @plsc.parallel_loop(0, _CLEN // 16, 1, unroll=8)
        def _(v):
            sl = pl.ds(v * 16, 16)
            g = plsc.load_gather(xstage, [rowc_v[sl], colc_v[sl]])
            idxbuf[sl] = g + offc_v[sl]
        pltpu.sync_copy(
            idxbuf, idx_hbm.at[pl.ds((wid * _NCHUNK + blk) * _CLEN, _CLEN)])

    def rd(cb, b):
        return pltpu.async_copy(
            tt_hbm.at[pl.ds(0, 16), pl.ds(cb * _DTW, _DTW)], stages[b],
            rsems[b])

    def rd_wait(b):
        pltpu.make_async_copy(
            tt_hbm.at[pl.ds(0, 16), pl.ds(0, _DTW)], stages[b],
            rsems[b]).wait()

    def wr(cb, b):
        return pltpu.async_copy(
            rows[b], flat_hbm.at[pl.ds(cb * _DTW * 16, _DTW * 16)], wsems[b])

    def wr_wait(b):
        pltpu.make_async_copy(
            rows[b], flat_hbm.at[pl.ds(0, _DTW * 16)], wsems[b]).wait()

    def transpose_block(stage, rbuf, width):
        @plsc.parallel_loop(0, width, 1, unroll=8)
        def _(j):
            g = plsc.load_gather(stage, [row16, row16 * 0 + j])
            rbuf[pl.ds(j * 16, 16)] = g

    cb0 = wid

    @pl.when(cb0 < _NCB)
    def _():
        rd(cb0, 0)

    def dstep(k, _):
        cb = k * _NW + wid
        cbn = cb + _NW

        def body(b, nb):
            @pl.when(cbn < _NCB)
            def _():
                rd(cbn, nb)

            @pl.when(cb < _NCB)
            def _():
                rd_wait(b)

                @pl.when(k >= 2)
                def _():
                    wr_wait(b)
                transpose_block(stages[b], rows[b], _DTW)
                wr(cb, b)

        @pl.when(k % 2 == 0)
        def _():
            body(0, 1)

        @pl.when(k % 2 == 1)
        def _():
            body(1, 0)
        return 0
    lax.fori_loop(0, _DSTEPS, dstep, 0)

    wr_wait(0)
    wr_wait(1)

    @pl.when(wid == _NW - 2)
    def _():
        pltpu.sync_copy(
            tt_hbm.at[pl.ds(0, 16), pl.ds(_TAIL_TC * 128, 512)], res_s)
        transpose_block(res_s, res_r, 512)
        pltpu.sync_copy(
            res_r, flat_hbm.at[pl.ds(_TAIL_TC * 128 * 16, 8192)])

    @pl.when(wid == _NW - 1)
    def _():
        for c in range(16):
            pltpu.sync_copy(
                tt_hbm.at[c, pl.ds((_NTC - 1) * 128, 76)], tail_s.at[c])

        def tailbody(j, _):
            g = plsc.load_gather(tail_s, [row16, row16 * 0 + j])
            tail_r[pl.ds(j * 16, 16)] = g
            return 0
        lax.fori_loop(0, 76, tailbody, 0)
        pltpu.sync_copy(
            tail_r, flat_hbm.at[pl.ds((_NTC - 1) * 128 * 16, 1216)])


@functools.partial(
    pl.kernel,
    mesh=_mesh,
    out_type=jax.ShapeDtypeStruct((_NF * 16, 8, 2048), jnp.float32),
    compiler_params=pltpu.CompilerParams(use_tc_tiling_on_sc=False,
                                         needs_layout_passes=False),
    scratch_types=[
        pltpu.VMEM((_BPW * _NF,), jnp.int32),
        pltpu.VMEM((_CLEN, _E), jnp.float32),
        pltpu.VMEM((_CLEN, _E), jnp.float32),
        pltpu.VMEM((8, _CB), jnp.float32),
        pltpu.VMEM((8, _CB), jnp.float32),
        pltpu.SemaphoreType.DMA,
        pltpu.SemaphoreType.DMA,
        pltpu.SemaphoreType.DMA,
        pltpu.SemaphoreType.DMA,
        pltpu.SemaphoreType.DMA,
    ],
)
def _gather(idx_hbm, rows_hbm, out_hbm,
            idx_v, gb0, gb1, wb0, wb1, sg0, sg1, sw0, sw1, ssem):
    wid = lax.axis_index("s") * 2 + lax.axis_index("c")
    r26 = lax.iota(jnp.int32, 16) * _NF
    zero16 = lax.iota(jnp.int32, 16) * 0
    gbufs = (gb0, gb1)
    gsems = (sg0, sg1)
    wbufs = (wb0, wb1)
    wsems = (sw0, sw1)

    pltpu.sync_copy(
        idx_hbm.at[pl.ds(wid * _BPW * _NF, _BPW * _NF)], idx_v)

    def start_chunk(blk, b):
        return pltpu.async_copy(
            rows_hbm.at[idx_v.at[pl.ds(blk * _CLEN, _CLEN)]], gbufs[b],
            gsems[b])

    def emit_chunk(blk, b):
        gbuf = gbufs[b]
        tcb = wid * _NCHUNK + blk
        u_base = tcb // 16
        col = (tcb % 16) * _CB

        def tbody(t, _):
            f = t >> 1
            tr = t & 1

            def wait_par(wb, ws):
                pltpu.make_async_copy(
                    wb, out_hbm.at[0, pl.ds(0, 8), pl.ds(0, _CB)], ws).wait()

            @pl.when((t >= 2) & (tr == 0))
            def _():
                wait_par(wbufs[0], wsems[0])

            @pl.when((t >= 2) & (tr == 1))
            def _():
                wait_par(wbufs[1], wsems[1])

            def build(wb):
                @plsc.parallel_loop(0, 8, 1, unroll=4)
                def _(dbb):
                    rowvec = r26 + (dbb * (16 * _NF) + f)
                    for cq in range(8):
                        colvec = zero16 + (tr * 8 + cq)
                        g = plsc.load_gather(gbuf, [rowvec, colvec])
                        wb[cq, pl.ds(dbb * 16, 16)] = g

            u = f * 16 + tr * 8 + u_base

            @pl.when(tr == 0)
            def _():
                build(wbufs[0])
                pltpu.async_copy(
                    wbufs[0], out_hbm.at[u, pl.ds(0, 8), pl.ds(col, _CB)],
                    wsems[0])

            @pl.when(tr == 1)
            def _():
                build(wbufs[1])
                pltpu.async_copy(
                    wbufs[1], out_hbm.at[u, pl.ds(0, 8), pl.ds(col, _CB)],
                    wsems[1])
            return 0
        lax.fori_loop(0, _NF * 2, tbody, 0)
        pltpu.make_async_copy(
            wbufs[0], out_hbm.at[0, pl.ds(0, 8), pl.ds(0, _CB)],
            wsems[0]).wait()
        pltpu.make_async_copy(
            wbufs[1], out_hbm.at[0, pl.ds(0, 8), pl.ds(0, _CB)],
            wsems[1]).wait()

    h = start_chunk(0, 0)
    for blk in range(_NCHUNK):
        if blk + 1 < _NCHUNK:
            hn = start_chunk(blk + 1, (blk + 1) % 2)
        h.wait()
        emit_chunk(blk, blk % 2)
        if blk + 1 < _NCHUNK:
            h = hn


_NUNIT = _NF * 16
_UPW = _NUNIT // _NW


@functools.partial(
    pl.kernel,
    mesh=_mesh,
    out_type=jax.ShapeDtypeStruct((_NF, _E, _B), jnp.float32),
    compiler_params=_TILED,
    scratch_types=[
        pltpu.VMEM((8, 2048), jnp.float32),
        pltpu.VMEM((8, 2048), jnp.float32),
        pltpu.SemaphoreType.DMA,
        pltpu.SemaphoreType.DMA,
        pltpu.SemaphoreType.DMA,
        pltpu.SemaphoreType.DMA,
    ],
)
def _format(o1d_hbm, out_hbm, st0, st1, rs0, rs1, ws0, ws1):
    wid = lax.axis_index("s") * 2 + lax.axis_index("c")
    stages = (st0, st1)
    rsems = (rs0, rs1)
    wsems = (ws0, ws1)

    def rd(u, b):
        base = u * 16384
        for cq in range(8):
            pltpu.async_copy(
                o1d_hbm.at[pl.ds(base + cq * 2048, 2048)],
                stages[b].at[cq], rsems[b])

    def rd_wait(b):
        for cq in range(8):
            pltpu.make_async_copy(
                o1d_hbm.at[pl.ds(0, 2048)], stages[b].at[cq],
                rsems[b]).wait()

    def wr(u, b):
        f = u // 16
        r8 = u % 16
        tr = r8 // 8
        e = r8 % 8
        return pltpu.async_copy(
            stages[b],
            out_hbm.at[f, pl.ds(tr * 8, 8), pl.ds(e * 2048, 2048)],
            wsems[b])

    def wr_wait(b):
        pltpu.make_async_copy(
            stages[b], out_hbm.at[0, pl.ds(0, 8), pl.ds(0, 2048)],
            wsems[b]).wait()

    rd(wid * _UPW, 0)
    for i in range(_UPW):
        u = wid * _UPW + i
        b = i % 2
        nb = (i + 1) % 2
        if i + 1 < _UPW:
            if i >= 1:
                wr_wait(nb)
            rd(u + 1, nb)
        rd_wait(b)
        wr(u, b)
    wr_wait(0)
    wr_wait(1)


def kernel(x, table):
    tt = jnp.transpose(table)
    xt = jnp.transpose(x)
    rowc = jnp.asarray(_ROWC_NP)
    colc = jnp.asarray(_COLC_NP)
    offc = jnp.asarray(_OFFC_NP)
    flat, idx = _prep(tt, xt, rowc, colc, offc)
    rows2d = flat.reshape(_VP, _E)
    o3 = _gather(idx, rows2d)
    o1d = o3.reshape(_NUNIT * 16384)
    out = _format(o1d)
    return jnp.transpose(out, (2, 0, 1))


# speedup vs baseline: 2.2420x; 1.0227x over previous
import functools

import jax
import jax.numpy as jnp
import numpy as np
from jax import lax
from jax.experimental import pallas as pl
from jax.experimental.pallas import tpu as pltpu
from jax.experimental.pallas import tpu_sc as plsc

_FIELD_DIM = 38462
_NF = 26
_B = 16384
_E = 16
_V = _FIELD_DIM * _NF
_VP = 1000064
_NTC = _VP // 128
_FLAT = _B * _NF
_NW = 32
_BPW = _B // _NW
_CB = 128
_CLEN = _CB * _NF
_NCHUNK = _BPW // _CB

_TCPB = 8
_DTW = _TCPB * 128
_NCB = (_NTC - 1) // _TCPB
_DSTEPS = (_NCB + _NW - 1) // _NW
_TAIL_TC = _NCB * _TCPB

_JJ = np.arange(_CLEN, dtype=np.int32)
_ROWC_NP = _JJ % _NF
_COLC_NP = _JJ // _NF
_OFFC_NP = (_JJ % _NF) * _FIELD_DIM

_mesh = plsc.VectorSubcoreMesh(core_axis_name="c", subcore_axis_name="s")
_TILED = pltpu.CompilerParams(use_tc_tiling_on_sc=True,
                              needs_layout_passes=False)


@functools.partial(
    pl.kernel,
    mesh=_mesh,
    out_type=(jax.ShapeDtypeStruct((_VP * _E,), jnp.float32),
              jax.ShapeDtypeStruct((_FLAT,), jnp.int32)),
    compiler_params=_TILED,
    scratch_types=[
        pltpu.VMEM((16, _DTW), jnp.float32),
        pltpu.VMEM((16, _DTW), jnp.float32),
        pltpu.VMEM((_DTW * 16,), jnp.float32),
        pltpu.VMEM((_DTW * 16,), jnp.float32),
        pltpu.VMEM((_NF, _CB), jnp.int32),
        pltpu.VMEM((_CLEN,), jnp.int32),
        pltpu.VMEM((_CLEN,), jnp.int32),
        pltpu.VMEM((_CLEN,), jnp.int32),
        pltpu.VMEM((_CLEN,), jnp.int32),
        pltpu.VMEM((16, 512), jnp.float32),
        pltpu.VMEM((8192,), jnp.float32),
        pltpu.VMEM((16, 76), jnp.float32),
        pltpu.VMEM((1216,), jnp.float32),
        pltpu.SemaphoreType.DMA,
        pltpu.SemaphoreType.DMA,
        pltpu.SemaphoreType.DMA,
        pltpu.SemaphoreType.DMA,
        pltpu.SemaphoreType.DMA,
    ],
)
def _prep(tt_hbm, xt_hbm, rowc_hbm, colc_hbm, offc_hbm,
          flat_hbm, idx_hbm,
          stage0, stage1, rows0, rows1, xstage, idxbuf,
          rowc_v, colc_v, offc_v, res_s, res_r, tail_s, tail_r,
          rs0, rs1, ws0, ws1, ssem):
    wid = lax.axis_index("s") * 2 + lax.axis_index("c")
    row16 = lax.iota(jnp.int32, 16)
    zero16 = row16 * 0
    stages = (stage0, stage1)
    rows = (rows0, rows1)
    rsems = (rs0, rs1)
    wsems = (ws0, ws1)

    pltpu.async_copy(rowc_hbm, rowc_v, ssem)
    pltpu.async_copy(colc_hbm, colc_v, ssem)
    pltpu.async_copy(offc_hbm, offc_v, ssem).wait()
    pltpu.make_async_copy(rowc_hbm, rowc_v, ssem).wait()
    pltpu.make_async_copy(colc_hbm, colc_v, ssem).wait()
    for blk in range(_NCHUNK):
        bsl = pl.ds((wid * _NCHUNK + blk) * _CB, _CB)
        pltpu.async_copy(xt_hbm.at[pl.ds(0, 8), bsl],
                         xstage.at[pl.ds(0, 8)], ssem)
        pltpu.async_copy(xt_hbm.at[pl.ds(8, 8), bsl],
                         xstage.at[pl.ds(8, 8)], ssem)
        pltpu.async_copy(xt_hbm.at[pl.ds(16, 8), bsl],
                         xstage.at[pl.ds(16, 8)], ssem)
        pltpu.async_copy(xt_hbm.at[pl.ds(24, 2), bsl],
                         xstage.at[pl.ds(24, 2)], ssem).wait()
        pltpu.make_async_copy(xt_hbm.at[pl.ds(0, 8), bsl],
                              xstage.at[pl.ds(0, 8)], ssem).wait()
        pltpu.make_async_copy(xt_hbm.at[pl.ds(8, 8), bsl],
                              xstage.at[pl.ds(8, 8)], ssem).wait()
        pltpu.make_async_copy(xt_hbm.at[pl.ds(16, 8), bsl],
                              xstage.at[pl.ds(16, 8)], ssem).wait()

        @plsc.parallel_loop(0, _CLEN // 16, 1, unroll=16)
        def _(v):
            sl = pl.ds(v * 16, 16)
            g = plsc.load_gather(xstage, [rowc_v[sl], colc_v[sl]])
            idxbuf[sl] = g + offc_v[sl]
        pltpu.sync_copy(
            idxbuf, idx_hbm.at[pl.ds((wid * _NCHUNK + blk) * _CLEN, _CLEN)])

    def rd(cb, b):
        return pltpu.async_copy(
            tt_hbm.at[pl.ds(0, 16), pl.ds(cb * _DTW, _DTW)], stages[b],
            rsems[b])

    def rd_wait(b):
        pltpu.make_async_copy(
            tt_hbm.at[pl.ds(0, 16), pl.ds(0, _DTW)], stages[b],
            rsems[b]).wait()

    def wr(cb, b):
        return pltpu.async_copy(
            rows[b], flat_hbm.at[pl.ds(cb * _DTW * 16, _DTW * 16)], wsems[b])

    def wr_wait(b):
        pltpu.make_async_copy(
            rows[b], flat_hbm.at[pl.ds(0, _DTW * 16)], wsems[b]).wait()

    def transpose_block(stage, rbuf, width):
        @plsc.parallel_loop(0, width, 1, unroll=16)
        def _(j):
            g = plsc.load_gather(stage, [row16, zero16 + j])
            rbuf[pl.ds(j * 16, 16)] = g

    cb0 = wid

    @pl.when(cb0 < _NCB)
    def _():
        rd(cb0, 0)

    def dstep(k, _):
        cb = k * _NW + wid
        cbn = cb + _NW

        def body(b, nb):
            @pl.when(cbn < _NCB)
            def _():
                rd(cbn, nb)

            @pl.when(cb < _NCB)
            def _():
                rd_wait(b)

                @pl.when(k >= 2)
                def _():
                    wr_wait(b)
                transpose_block(stages[b], rows[b], _DTW)
                wr(cb, b)

        @pl.when(k % 2 == 0)
        def _():
            body(0, 1)

        @pl.when(k % 2 == 1)
        def _():
            body(1, 0)
        return 0
    lax.fori_loop(0, _DSTEPS, dstep, 0)

    wr_wait(0)
    wr_wait(1)

    @pl.when(wid == _NW - 2)
    def _():
        pltpu.sync_copy(
            tt_hbm.at[pl.ds(0, 16), pl.ds(_TAIL_TC * 128, 512)], res_s)
        transpose_block(res_s, res_r, 512)
        pltpu.sync_copy(
            res_r, flat_hbm.at[pl.ds(_TAIL_TC * 128 * 16, 8192)])

    @pl.when(wid == _NW - 1)
    def _():
        for c in range(16):
            pltpu.sync_copy(
                tt_hbm.at[c, pl.ds((_NTC - 1) * 128, 76)], tail_s.at[c])

        @plsc.parallel_loop(0, 76, 1, unroll=4)
        def _(j):
            g = plsc.load_gather(tail_s, [row16, zero16 + j])
            tail_r[pl.ds(j * 16, 16)] = g
        pltpu.sync_copy(
            tail_r, flat_hbm.at[pl.ds((_NTC - 1) * 128 * 16, 1216)])


@functools.partial(
    pl.kernel,
    mesh=_mesh,
    out_type=jax.ShapeDtypeStruct((_NF * 16, 8, 2048), jnp.float32),
    compiler_params=pltpu.CompilerParams(use_tc_tiling_on_sc=False,
                                         needs_layout_passes=False),
    scratch_types=[
        pltpu.VMEM((_BPW * _NF,), jnp.int32),
        pltpu.VMEM((_CLEN, _E), jnp.float32),
        pltpu.VMEM((_CLEN, _E), jnp.float32),
        pltpu.VMEM((8, _CB), jnp.float32),
        pltpu.VMEM((8, _CB), jnp.float32),
        pltpu.SemaphoreType.DMA,
        pltpu.SemaphoreType.DMA,
        pltpu.SemaphoreType.DMA,
        pltpu.SemaphoreType.DMA,
        pltpu.SemaphoreType.DMA,
    ],
)
def _gather(idx_hbm, rows_hbm, out_hbm,
            idx_v, gb0, gb1, wb0, wb1, sg0, sg1, sw0, sw1, ssem):
    wid = lax.axis_index("s") * 2 + lax.axis_index("c")
    r26 = lax.iota(jnp.int32, 16) * _NF
    zero16 = lax.iota(jnp.int32, 16) * 0
    gbufs = (gb0, gb1)
    gsems = (sg0, sg1)
    wbufs = (wb0, wb1)
    wsems = (sw0, sw1)

    pltpu.sync_copy(
        idx_hbm.at[pl.ds(wid * _BPW * _NF, _BPW * _NF)], idx_v)

    def start_chunk(blk, b):
        return pltpu.async_copy(
            rows_hbm.at[idx_v.at[pl.ds(blk * _CLEN, _CLEN)]], gbufs[b],
            gsems[b])

    def emit_chunk(blk, b):
        gbuf = gbufs[b]
        tcb = wid * _NCHUNK + blk
        u_base = tcb // 16
        col = (tcb % 16) * _CB

        def tbody(t, _):
            f = t >> 1
            tr = t & 1

            def wait_par(wb, ws):
                pltpu.make_async_copy(
                    wb, out_hbm.at[0, pl.ds(0, 8), pl.ds(0, _CB)], ws).wait()

            @pl.when((t >= 2) & (tr == 0))
            def _():
                wait_par(wbufs[0], wsems[0])

            @pl.when((t >= 2) & (tr == 1))
            def _():
                wait_par(wbufs[1], wsems[1])

            def build(wb):
                @plsc.parallel_loop(0, 8, 1, unroll=4)
                def _(dbb):
                    rowvec = r26 + (dbb * (16 * _NF) + f)
                    for cq in range(8):
                        colvec = zero16 + (tr * 8 + cq)
                        g = plsc.load_gather(gbuf, [rowvec, colvec])
                        wb[cq, pl.ds(dbb * 16, 16)] = g

            u = f * 16 + tr * 8 + u_base

            @pl.when(tr == 0)
            def _():
                build(wbufs[0])
                pltpu.async_copy(
                    wbufs[0], out_hbm.at[u, pl.ds(0, 8), pl.ds(col, _CB)],
                    wsems[0])

            @pl.when(tr == 1)
            def _():
                build(wbufs[1])
                pltpu.async_copy(
                    wbufs[1], out_hbm.at[u, pl.ds(0, 8), pl.ds(col, _CB)],
                    wsems[1])
            return 0
        lax.fori_loop(0, _NF * 2, tbody, 0)
        pltpu.make_async_copy(
            wbufs[0], out_hbm.at[0, pl.ds(0, 8), pl.ds(0, _CB)],
            wsems[0]).wait()
        pltpu.make_async_copy(
            wbufs[1], out_hbm.at[0, pl.ds(0, 8), pl.ds(0, _CB)],
            wsems[1]).wait()

    h = start_chunk(0, 0)
    for blk in range(_NCHUNK):
        if blk + 1 < _NCHUNK:
            hn = start_chunk(blk + 1, (blk + 1) % 2)
        h.wait()
        emit_chunk(blk, blk % 2)
        if blk + 1 < _NCHUNK:
            h = hn


_NUNIT = _NF * 16
_UPW = _NUNIT // _NW


@functools.partial(
    pl.kernel,
    mesh=_mesh,
    out_type=jax.ShapeDtypeStruct((_NF, _E, _B), jnp.float32),
    compiler_params=_TILED,
    scratch_types=[
        pltpu.VMEM((8, 2048), jnp.float32),
        pltpu.VMEM((8, 2048), jnp.float32),
        pltpu.SemaphoreType.DMA,
        pltpu.SemaphoreType.DMA,
        pltpu.SemaphoreType.DMA,
        pltpu.SemaphoreType.DMA,
    ],
)
def _format(o1d_hbm, out_hbm, st0, st1, rs0, rs1, ws0, ws1):
    wid = lax.axis_index("s") * 2 + lax.axis_index("c")
    stages = (st0, st1)
    rsems = (rs0, rs1)
    wsems = (ws0, ws1)

    def rd(u, b):
        base = u * 16384
        for cq in range(8):
            pltpu.async_copy(
                o1d_hbm.at[pl.ds(base + cq * 2048, 2048)],
                stages[b].at[cq], rsems[b])

    def rd_wait(b):
        for cq in range(8):
            pltpu.make_async_copy(
                o1d_hbm.at[pl.ds(0, 2048)], stages[b].at[cq],
                rsems[b]).wait()

    def wr(u, b):
        f = u // 16
        r8 = u % 16
        tr = r8 // 8
        e = r8 % 8
        return pltpu.async_copy(
            stages[b],
            out_hbm.at[f, pl.ds(tr * 8, 8), pl.ds(e * 2048, 2048)],
            wsems[b])

    def wr_wait(b):
        pltpu.make_async_copy(
            stages[b], out_hbm.at[0, pl.ds(0, 8), pl.ds(0, 2048)],
            wsems[b]).wait()

    rd(wid * _UPW, 0)
    for i in range(_UPW):
        u = wid * _UPW + i
        b = i % 2
        nb = (i + 1) % 2
        if i + 1 < _UPW:
            if i >= 1:
                wr_wait(nb)
            rd(u + 1, nb)
        rd_wait(b)
        wr(u, b)
    wr_wait(0)
    wr_wait(1)


def kernel(x, table):
    tt = jnp.transpose(table)
    xt = jnp.transpose(x)
    rowc = jnp.asarray(_ROWC_NP)
    colc = jnp.asarray(_COLC_NP)
    offc = jnp.asarray(_OFFC_NP)
    flat, idx = _prep(tt, xt, rowc, colc, offc)
    rows2d = flat.reshape(_VP, _E)
    o3 = _gather(idx, rows2d)
    o1d = o3.reshape(_NUNIT * 16384)
    out = _format(o1d)
    return jnp.transpose(out, (2, 0, 1))


# speedup vs baseline: 2.2546x; 1.0056x over previous
import functools

import jax
import jax.numpy as jnp
import numpy as np
from jax import lax
from jax.experimental import pallas as pl
from jax.experimental.pallas import tpu as pltpu
from jax.experimental.pallas import tpu_sc as plsc

_FIELD_DIM = 38462
_NF = 26
_B = 16384
_E = 16
_V = _FIELD_DIM * _NF
_VP = 1000064
_NTC = _VP // 128
_FLAT = _B * _NF
_NW = 32
_BPW = _B // _NW
_CB = 128
_CLEN = _CB * _NF
_NCHUNK = _BPW // _CB
_EP = 16
_SP = 1032

_TCPB = 8
_DTW = _TCPB * 128
_NCB = (_NTC - 1) // _TCPB
_DSTEPS = (_NCB + _NW - 1) // _NW
_TAIL_TC = _NCB * _TCPB

_JJ = np.arange(_CLEN, dtype=np.int32)
_ROWC_NP = _JJ % _NF
_COLC_NP = _JJ // _NF
_OFFC_NP = (_JJ % _NF) * _FIELD_DIM

_mesh = plsc.VectorSubcoreMesh(core_axis_name="c", subcore_axis_name="s")
_TILED = pltpu.CompilerParams(use_tc_tiling_on_sc=True,
                              needs_layout_passes=False)


@functools.partial(
    pl.kernel,
    mesh=_mesh,
    out_type=(jax.ShapeDtypeStruct((_VP * _EP,), jnp.float32),
              jax.ShapeDtypeStruct((_FLAT,), jnp.int32)),
    compiler_params=_TILED,
    scratch_types=[
        pltpu.VMEM((16, _SP), jnp.float32),
        pltpu.VMEM((16, _SP), jnp.float32),
        pltpu.VMEM((_DTW * _EP,), jnp.float32),
        pltpu.VMEM((_DTW * _EP,), jnp.float32),
        pltpu.VMEM((_NF, _CB), jnp.int32),
        pltpu.VMEM((_CLEN,), jnp.int32),
        pltpu.VMEM((_CLEN,), jnp.int32),
        pltpu.VMEM((_CLEN,), jnp.int32),
        pltpu.VMEM((_CLEN,), jnp.int32),
        pltpu.VMEM((16, 520), jnp.float32),
        pltpu.VMEM((512 * _EP,), jnp.float32),
        pltpu.VMEM((16, 80), jnp.float32),
        pltpu.VMEM((76 * _EP,), jnp.float32),
        pltpu.SemaphoreType.DMA,
        pltpu.SemaphoreType.DMA,
        pltpu.SemaphoreType.DMA,
        pltpu.SemaphoreType.DMA,
        pltpu.SemaphoreType.DMA,
    ],
)
def _prep(tt_hbm, xt_hbm, rowc_hbm, colc_hbm, offc_hbm,
          flat_hbm, idx_hbm,
          stage0, stage1, rows0, rows1, xstage, idxbuf,
          rowc_v, colc_v, offc_v, res_s, res_r, tail_s, tail_r,
          rs0, rs1, ws0, ws1, ssem):
    wid = lax.axis_index("s") * 2 + lax.axis_index("c")
    row16 = lax.iota(jnp.int32, 16)
    zero16 = row16 * 0
    stages = (stage0, stage1)
    rows = (rows0, rows1)
    rsems = (rs0, rs1)
    wsems = (ws0, ws1)

    pltpu.async_copy(rowc_hbm, rowc_v, ssem)
    pltpu.async_copy(colc_hbm, colc_v, ssem)
    pltpu.async_copy(offc_hbm, offc_v, ssem).wait()
    pltpu.make_async_copy(rowc_hbm, rowc_v, ssem).wait()
    pltpu.make_async_copy(colc_hbm, colc_v, ssem).wait()
    for blk in range(_NCHUNK):
        bsl = pl.ds((wid * _NCHUNK + blk) * _CB, _CB)
        pltpu.async_copy(xt_hbm.at[pl.ds(0, 8), bsl],
                         xstage.at[pl.ds(0, 8)], ssem)
        pltpu.async_copy(xt_hbm.at[pl.ds(8, 8), bsl],
                         xstage.at[pl.ds(8, 8)], ssem)
        pltpu.async_copy(xt_hbm.at[pl.ds(16, 8), bsl],
                         xstage.at[pl.ds(16, 8)], ssem)
        pltpu.async_copy(xt_hbm.at[pl.ds(24, 2), bsl],
                         xstage.at[pl.ds(24, 2)], ssem).wait()
        pltpu.make_async_copy(xt_hbm.at[pl.ds(0, 8), bsl],
                              xstage.at[pl.ds(0, 8)], ssem).wait()
        pltpu.make_async_copy(xt_hbm.at[pl.ds(8, 8), bsl],
                              xstage.at[pl.ds(8, 8)], ssem).wait()
        pltpu.make_async_copy(xt_hbm.at[pl.ds(16, 8), bsl],
                              xstage.at[pl.ds(16, 8)], ssem).wait()

        @plsc.parallel_loop(0, _CLEN // 16, 1, unroll=16)
        def _(v):
            sl = pl.ds(v * 16, 16)
            g = plsc.load_gather(xstage, [rowc_v[sl], colc_v[sl]])
            idxbuf[sl] = g + offc_v[sl]
        pltpu.sync_copy(
            idxbuf, idx_hbm.at[pl.ds((wid * _NCHUNK + blk) * _CLEN, _CLEN)])

    def rd(cb, b):
        return pltpu.async_copy(
            tt_hbm.at[pl.ds(0, 16), pl.ds(cb * _DTW, _DTW)],
            stages[b].at[pl.ds(0, 16), pl.ds(0, _DTW)], rsems[b])

    def rd_wait(b):
        pltpu.make_async_copy(
            tt_hbm.at[pl.ds(0, 16), pl.ds(0, _DTW)],
            stages[b].at[pl.ds(0, 16), pl.ds(0, _DTW)], rsems[b]).wait()

    def wr(cb, b):
        return pltpu.async_copy(
            rows[b], flat_hbm.at[pl.ds(cb * _DTW * _EP, _DTW * _EP)], wsems[b])

    def wr_wait(b):
        pltpu.make_async_copy(
            rows[b], flat_hbm.at[pl.ds(0, _DTW * _EP)], wsems[b]).wait()

    def transpose_block(stage, rbuf, width):
        @plsc.parallel_loop(0, width, 1, unroll=16)
        def _(j):
            g = plsc.load_gather(stage, [row16, zero16 + j])
            rbuf[pl.ds(j * _EP, 16)] = g

    cb0 = wid

    @pl.when(cb0 < _NCB)
    def _():
        rd(cb0, 0)

    def dstep(k, _):
        cb = k * _NW + wid
        cbn = cb + _NW

        def body(b, nb):
            @pl.when(cbn < _NCB)
            def _():
                rd(cbn, nb)

            @pl.when(cb < _NCB)
            def _():
                rd_wait(b)

                @pl.when(k >= 2)
                def _():
                    wr_wait(b)
                transpose_block(stages[b], rows[b], _DTW)
                wr(cb, b)

        @pl.when(k % 2 == 0)
        def _():
            body(0, 1)

        @pl.when(k % 2 == 1)
        def _():
            body(1, 0)
        return 0
    lax.fori_loop(0, _DSTEPS, dstep, 0)

    wr_wait(0)
    wr_wait(1)

    @pl.when(wid == _NW - 2)
    def _():
        pltpu.sync_copy(
            tt_hbm.at[pl.ds(0, 16), pl.ds(_TAIL_TC * 128, 512)],
            res_s.at[pl.ds(0, 16), pl.ds(0, 512)])
        transpose_block(res_s, res_r, 512)
        pltpu.sync_copy(
            res_r, flat_hbm.at[pl.ds(_TAIL_TC * 128 * _EP, 512 * _EP)])

    @pl.when(wid == _NW - 1)
    def _():
        for c in range(16):
            pltpu.sync_copy(
                tt_hbm.at[c, pl.ds((_NTC - 1) * 128, 76)],
                tail_s.at[c, pl.ds(0, 76)])

        @plsc.parallel_loop(0, 76, 1, unroll=4)
        def _(j):
            g = plsc.load_gather(tail_s, [row16, zero16 + j])
            tail_r[pl.ds(j * _EP, 16)] = g
        pltpu.sync_copy(
            tail_r, flat_hbm.at[pl.ds((_NTC - 1) * 128 * _EP, 76 * _EP)])


@functools.partial(
    pl.kernel,
    mesh=_mesh,
    out_type=jax.ShapeDtypeStruct((_NF * 16, 8, 2048), jnp.float32),
    compiler_params=pltpu.CompilerParams(use_tc_tiling_on_sc=False,
                                         needs_layout_passes=False),
    scratch_types=[
        pltpu.VMEM((_BPW * _NF,), jnp.int32),
        pltpu.VMEM((_CLEN, _E), jnp.float32),
        pltpu.VMEM((_CLEN, _E), jnp.float32),
        pltpu.VMEM((8, _CB), jnp.float32),
        pltpu.VMEM((8, _CB), jnp.float32),
        pltpu.SemaphoreType.DMA,
        pltpu.SemaphoreType.DMA,
        pltpu.SemaphoreType.DMA,
        pltpu.SemaphoreType.DMA,
        pltpu.SemaphoreType.DMA,
    ],
)
def _gather(idx_hbm, rows_hbm, out_hbm,
            idx_v, gb0, gb1, wb0, wb1, sg0, sg1, sw0, sw1, ssem):
    wid = lax.axis_index("s") * 2 + lax.axis_index("c")
    r26 = lax.iota(jnp.int32, 16) * _NF
    zero16 = lax.iota(jnp.int32, 16) * 0
    gbufs = (gb0, gb1)
    gsems = (sg0, sg1)
    wbufs = (wb0, wb1)
    wsems = (sw0, sw1)

    pltpu.sync_copy(
        idx_hbm.at[pl.ds(wid * _BPW * _NF, _BPW * _NF)], idx_v)

    def start_chunk(blk, b):
        return pltpu.async_copy(
            rows_hbm.at[idx_v.at[pl.ds(blk * _CLEN, _CLEN)]], gbufs[b],
            gsems[b])

    def emit_chunk(blk, b):
        gbuf = gbufs[b]
        tcb = wid * _NCHUNK + blk
        u_base = tcb // 16
        col = (tcb % 16) * _CB

        def tbody(t, _):
            f = t >> 1
            tr = t & 1

            def wait_par(wb, ws):
                pltpu.make_async_copy(
                    wb, out_hbm.at[0, pl.ds(0, 8), pl.ds(0, _CB)], ws).wait()

            @pl.when((t >= 2) & (tr == 0))
            def _():
                wait_par(wbufs[0], wsems[0])

            @pl.when((t >= 2) & (tr == 1))
            def _():
                wait_par(wbufs[1], wsems[1])

            def build(wb):
                @plsc.parallel_loop(0, 8, 1, unroll=4)
                def _(dbb):
                    rowvec = r26 + (dbb * (16 * _NF) + f)
                    for cq in range(8):
                        colvec = zero16 + (tr * 8 + cq)
                        g = plsc.load_gather(gbuf, [rowvec, colvec])
                        wb[cq, pl.ds(dbb * 16, 16)] = g

            u = f * 16 + tr * 8 + u_base

            @pl.when(tr == 0)
            def _():
                build(wbufs[0])
                pltpu.async_copy(
                    wbufs[0], out_hbm.at[u, pl.ds(0, 8), pl.ds(col, _CB)],
                    wsems[0])

            @pl.when(tr == 1)
            def _():
                build(wbufs[1])
                pltpu.async_copy(
                    wbufs[1], out_hbm.at[u, pl.ds(0, 8), pl.ds(col, _CB)],
                    wsems[1])
            return 0
        lax.fori_loop(0, _NF * 2, tbody, 0)
        pltpu.make_async_copy(
            wbufs[0], out_hbm.at[0, pl.ds(0, 8), pl.ds(0, _CB)],
            wsems[0]).wait()
        pltpu.make_async_copy(
            wbufs[1], out_hbm.at[0, pl.ds(0, 8), pl.ds(0, _CB)],
            wsems[1]).wait()

    h = start_chunk(0, 0)
    for blk in range(_NCHUNK):
        if blk + 1 < _NCHUNK:
            hn = start_chunk(blk + 1, (blk + 1) % 2)
        h.wait()
        emit_chunk(blk, blk % 2)
        if blk + 1 < _NCHUNK:
            h = hn


_NUNIT = _NF * 16
_UPW = _NUNIT // _NW


@functools.partial(
    pl.kernel,
    mesh=_mesh,
    out_type=jax.ShapeDtypeStruct((_NF, _E, _B), jnp.float32),
    compiler_params=_TILED,
    scratch_types=[
        pltpu.VMEM((8, 2048), jnp.float32),
        pltpu.VMEM((8, 2048), jnp.float32),
        pltpu.SemaphoreType.DMA,
        pltpu.SemaphoreType.DMA,
        pltpu.SemaphoreType.DMA,
        pltpu.SemaphoreType.DMA,
    ],
)
def _format(o1d_hbm, out_hbm, st0, st1, rs0, rs1, ws0, ws1):
    wid = lax.axis_index("s") * 2 + lax.axis_index("c")
    stages = (st0, st1)
    rsems = (rs0, rs1)
    wsems = (ws0, ws1)

    def rd(u, b):
        base = u * 16384
        for cq in range(8):
            pltpu.async_copy(
                o1d_hbm.at[pl.ds(base + cq * 2048, 2048)],
                stages[b].at[cq], rsems[b])

    def rd_wait(b):
        for cq in range(8):
            pltpu.make_async_copy(
                o1d_hbm.at[pl.ds(0, 2048)], stages[b].at[cq],
                rsems[b]).wait()

    def wr(u, b):
        f = u // 16
        r8 = u % 16
        tr = r8 // 8
        e = r8 % 8
        return pltpu.async_copy(
            stages[b],
            out_hbm.at[f, pl.ds(tr * 8, 8), pl.ds(e * 2048, 2048)],
            wsems[b])

    def wr_wait(b):
        pltpu.make_async_copy(
            stages[b], out_hbm.at[0, pl.ds(0, 8), pl.ds(0, 2048)],
            wsems[b]).wait()

    rd(wid * _UPW, 0)
    for i in range(_UPW):
        u = wid * _UPW + i
        b = i % 2
        nb = (i + 1) % 2
        if i + 1 < _UPW:
            if i >= 1:
                wr_wait(nb)
            rd(u + 1, nb)
        rd_wait(b)
        wr(u, b)
    wr_wait(0)
    wr_wait(1)


def kernel(x, table):
    tt = jnp.transpose(table)
    xt = jnp.transpose(x)
    rowc = jnp.asarray(_ROWC_NP)
    colc = jnp.asarray(_COLC_NP)
    offc = jnp.asarray(_OFFC_NP)
    flat, idx = _prep(tt, xt, rowc, colc, offc)
    rows2d = flat.reshape(_VP, _E)
    o3 = _gather(idx, rows2d)
    o1d = o3.reshape(_NUNIT * 16384)
    out = _format(o1d)
    return jnp.transpose(out, (2, 0, 1))
